# Initial kernel scaffold; baseline (speedup 1.0000x reference)
import jax
import jax.numpy as jnp
from jax.experimental import pallas as pl


def kernel(x, edge_attr, params, edge_index, batch):
    raise NotImplementedError("write your pallas kernel here")


# speedup vs baseline: 2.6622x; 2.6622x over previous
import functools
import jax
import jax.numpy as jnp
from jax import lax
from jax.experimental import pallas as pl
from jax.experimental.pallas import tpu as pltpu
from jax.experimental.pallas import tpu_sc as plsc

N = 50000
E = 800000
HID = 64
B = 16

NC = 2
NS = 16
LANES = 16

W = 128
EROWS = 6272
E_PAD = EROWS * W
N_PAD = 50176

R1 = 4
R2 = 4
NB1 = EROWS // NC // NS // R1
NB2 = EROWS // NS // R2
NPT = N_PAD // NS

BLK = 3136
GRID = N_PAD // BLK

_mesh = plsc.VectorSubcoreMesh(core_axis_name="c", subcore_axis_name="s")


def _relu(v):
    return jnp.maximum(v, 0.0)


def _bc(scalar, dtype=jnp.float32):
    return jnp.broadcast_to(jnp.asarray(scalar, dtype), (LANES,))


@functools.partial(
    pl.kernel,
    out_type=jax.ShapeDtypeStruct((NC, N_PAD, 8), jnp.float32),
    mesh=_mesh,
    compiler_params=pltpu.CompilerParams(
        needs_layout_passes=False, use_tc_tiling_on_sc=False),
    scratch_types=[
        pltpu.VMEM((R1, W), jnp.int32),
        pltpu.VMEM((R1 * W,), jnp.float32),
        pltpu.VMEM((W, 8), jnp.float32),
        pltpu.VMEM((128,), jnp.float32),
        pltpu.VMEM_SHARED((N_PAD, 8), jnp.float32),
    ],
)
def _k1(rowr, eap, wb, zeros8, out, rowv, eav, staging, wbv, acc):
    c = lax.axis_index("c")
    s = lax.axis_index("s")
    wid = c * NS + s
    iota = lax.iota(jnp.int32, LANES)
    ones = jnp.ones((LANES,), jnp.float32)
    zerov = jnp.zeros((LANES,), jnp.float32)

    pltpu.sync_copy(zeros8, acc.at[pl.ds(s * NPT, NPT)])

    pltpu.sync_copy(wb, wbv)
    wvec = wbv[pl.ds(0, LANES)]

    def _splat(i):
        oh = (iota == jnp.full((LANES,), i, jnp.int32)).astype(jnp.float32)
        return _bc(jnp.sum(wvec * oh))

    w3 = [_splat(h) for h in range(3)]
    b3 = [_splat(3 + h) for h in range(3)]

    for k in range(W // LANES):
        ir0 = iota + k * LANES
        for f in range(4, 8):
            plsc.store_scatter(staging, [ir0, jnp.full((LANES,), f, jnp.int32)], zerov)

    plsc.subcore_barrier()

    tile_row0 = wid * (R1 * NB1)
    tile_edge0 = tile_row0 * W

    def big(g, _):
        r0 = tile_row0 + g * R1
        pltpu.sync_copy(rowr.at[pl.ds(r0, R1)], rowv)
        pltpu.sync_copy(eap.at[pl.ds(tile_edge0 + g * (R1 * W), R1 * W)], eav)

        def perrow(j, _):
            def pergrp(k, _):
                eg = eav[pl.ds(j * W + k * LANES, LANES)]
                ir = iota + _bc(k * LANES, jnp.int32)
                plsc.store_scatter(staging, [ir, jnp.full((LANES,), 0, jnp.int32)], ones)
                for h in range(3):
                    ee = jnp.maximum(eg * w3[h] + b3[h], zerov)
                    plsc.store_scatter(
                        staging, [ir, jnp.full((LANES,), 1 + h, jnp.int32)], ee)
                return 0

            lax.fori_loop(0, W // LANES, pergrp, 0)
            pltpu.sync_copy(staging, acc.at[rowv.at[j]], add=True)
            return 0

        lax.fori_loop(0, R1, perrow, 0)
        return 0

    lax.fori_loop(0, NB1, big, 0)
    plsc.subcore_barrier()
    pltpu.sync_copy(acc.at[pl.ds(s * NPT, NPT)], out.at[c, pl.ds(s * NPT, NPT)])


@functools.partial(
    pl.kernel,
    out_type=jax.ShapeDtypeStruct((NC, N_PAD, 32), jnp.float32),
    mesh=_mesh,
    compiler_params=pltpu.CompilerParams(
        needs_layout_passes=False, use_tc_tiling_on_sc=False),
    scratch_types=[
        pltpu.VMEM((R2, W), jnp.int32),
        pltpu.VMEM((R2, W), jnp.int32),
        pltpu.VMEM((R2 * W,), jnp.float32),
        pltpu.VMEM((R2 * W, 32), jnp.float32),
        pltpu.VMEM_SHARED((N_PAD, 32), jnp.float32),
        pltpu.SemaphoreType.DMA,
    ],
)
def _k2(colr, rowr, eap, ubig, zeros32, out, colv, rowv, eav, gath, acc, sem):
    c = lax.axis_index("c")
    s = lax.axis_index("s")
    iota = lax.iota(jnp.int32, LANES)

    pltpu.sync_copy(zeros32, acc.at[pl.ds(s * NPT, NPT)])
    plsc.subcore_barrier()

    tile_row0 = s * (R2 * NB2)
    tile_edge0 = tile_row0 * W
    offv = _bc(c, jnp.int32) * jnp.full((LANES,), N_PAD, jnp.int32)

    def big(g, _):
        r0 = tile_row0 + g * R2
        pltpu.sync_copy(colr.at[pl.ds(r0, R2)], colv)
        pltpu.sync_copy(rowr.at[pl.ds(r0, R2)], rowv)
        pltpu.sync_copy(eap.at[pl.ds(tile_edge0 + g * (R2 * W), R2 * W)], eav)

        def addoff(gi, _):
            j = gi // (W // LANES)
            k = gi % (W // LANES)
            sl = pl.ds(k * LANES, LANES)
            colv[j, sl] = colv[j, sl] + offv
            return 0

        lax.fori_loop(0, R2 * (W // LANES), addoff, 0)

        cps = [
            pltpu.async_copy(ubig.at[colv.at[j]], gath.at[pl.ds(j * W, W)], sem)
            for j in range(R2)
        ]
        for cp in cps:
            cp.wait()

        def scale(gi, _):
            ev = eav[pl.ds(gi * LANES, LANES)]
            ir = iota + _bc(gi * LANES, jnp.int32)
            for col in range(32):
                ic = jnp.full((LANES,), col, jnp.int32)
                v = plsc.load_gather(gath, [ir, ic])
                plsc.store_scatter(gath, [ir, ic], v * ev)
            return 0

        lax.fori_loop(0, (R2 * W) // LANES, scale, 0)

        for j in range(R2):
            pltpu.sync_copy(gath.at[pl.ds(j * W, W)], acc.at[rowv.at[j]], add=True)
        return 0

    lax.fori_loop(0, NB2, big, 0)
    plsc.subcore_barrier()
    pltpu.sync_copy(acc.at[pl.ds(s * NPT, NPT)], out.at[c, pl.ds(s * NPT, NPT)])


def _row_spec(width):
    return pl.BlockSpec((BLK, width), lambda i: (i, 0))


def _full_spec(shape):
    return pl.BlockSpec(shape, lambda i: tuple(0 for _ in shape))


def _ka_body(pa, pb, x2, cA, ulo, uhi, aux):
    cnt = pa[:, 0:1] + pb[:, 0:1]
    dinv = 1.0 / jnp.maximum(cnt, 1.0)
    em0 = (pa[:, 1:2] + pb[:, 1:2]) * dinv
    em1 = (pa[:, 2:3] + pb[:, 2:3]) * dinv
    em2 = (pa[:, 3:4] + pb[:, 3:4]) * dinv
    x = x2[...]
    u = _relu(x * cA[0:1, :] + em0 * cA[1:2, :] + cA[2:3, :])
    ulo[...] = u[:, :32]
    uhi[...] = u[:, 32:]
    aux[...] = jnp.concatenate(
        [dinv, em1, em2, x, cnt, jnp.zeros((BLK, 3), jnp.float32)], axis=1)


def _ka(pa, pb, x2, cA):
    return pl.pallas_call(
        _ka_body,
        grid=(GRID,),
        in_specs=[_row_spec(8), _row_spec(8), _row_spec(1), _full_spec((3, HID))],
        out_specs=[_row_spec(32), _row_spec(32), _row_spec(8)],
        out_shape=[
            jax.ShapeDtypeStruct((N_PAD, 32), jnp.float32),
            jax.ShapeDtypeStruct((N_PAD, 32), jnp.float32),
            jax.ShapeDtypeStruct((N_PAD, 8), jnp.float32),
        ],
    )(pa, pb, x2, cA)


def _u_step(slo, shi, aux, w1t, cB, em_col):
    dinv = aux[:, 0:1]
    em = aux[:, em_col:em_col + 1]
    x = aux[:, 3:4]
    ua = jnp.concatenate([slo[...], shi[...]], axis=1) * dinv
    second = jnp.dot(ua, w1t[...], preferred_element_type=jnp.float32)
    return _relu(x * cB[0:1, :] + second + em * cB[1:2, :] + cB[2:3, :])


def _kb_body(slo, shi, aux, w1t, cB, ulo, uhi):
    u = _u_step(slo, shi, aux[...], w1t, cB, 1)
    ulo[...] = u[:, :32]
    uhi[...] = u[:, 32:]


def _kb(slo, shi, aux, w1t, cB):
    return pl.pallas_call(
        _kb_body,
        grid=(GRID,),
        in_specs=[_row_spec(32), _row_spec(32), _row_spec(8),
                  _full_spec((HID, HID)), _full_spec((3, HID))],
        out_specs=[_row_spec(32), _row_spec(32)],
        out_shape=[
            jax.ShapeDtypeStruct((N_PAD, 32), jnp.float32),
            jax.ShapeDtypeStruct((N_PAD, 32), jnp.float32),
        ],
    )(slo, shi, aux, w1t, cB)


def _kc_body(slo, shi, aux, batchp, w1t, cB, u2, s0, s1, den):
    auxv = aux[...]
    u = _u_step(slo, shi, auxv, w1t, cB, 2)
    u2[...] = u
    x = auxv[:, 3:4]
    bi = batchp[...]
    oh = (bi == lax.broadcasted_iota(jnp.int32, (1, B), 1)).astype(jnp.float32)
    dn = (((0,), (0,)), ((), ()))
    p0 = lax.dot_general(oh, u * (1.0 - x), dn, preferred_element_type=jnp.float32)
    p1 = lax.dot_general(oh, u * x, dn, preferred_element_type=jnp.float32)
    d0 = lax.dot_general(oh, 1.0 - x, dn, preferred_element_type=jnp.float32)
    d1 = lax.dot_general(oh, x, dn, preferred_element_type=jnp.float32)
    dd = jnp.concatenate([d0, d1, jnp.zeros((B, 6), jnp.float32)], axis=1)

    @pl.when(pl.program_id(0) == 0)
    def _init():
        s0[...] = jnp.zeros_like(s0)
        s1[...] = jnp.zeros_like(s1)
        den[...] = jnp.zeros_like(den)

    s0[...] += p0
    s1[...] += p1
    den[...] += dd


def _kc(slo, shi, aux, batchp, w1t, cB):
    return pl.pallas_call(
        _kc_body,
        grid=(GRID,),
        in_specs=[_row_spec(32), _row_spec(32), _row_spec(8), _row_spec(1),
                  _full_spec((HID, HID)), _full_spec((3, HID))],
        out_specs=[_row_spec(HID), _full_spec((B, HID)), _full_spec((B, HID)),
                   _full_spec((B, 8))],
        out_shape=[
            jax.ShapeDtypeStruct((N_PAD, HID), jnp.float32),
            jax.ShapeDtypeStruct((B, HID), jnp.float32),
            jax.ShapeDtypeStruct((B, HID), jnp.float32),
            jax.ShapeDtypeStruct((B, 8), jnp.float32),
        ],
    )(slo, shi, aux, batchp, w1t, cB)


def _kd_body(u2, aux, batchp, s0, s1, den, watu, watt, cD, q):
    hc0 = s0[...] / den[:, 0:1]
    hc1 = s1[...] / den[:, 1:2]
    bi = batchp[...]
    oh = (bi == lax.broadcasted_iota(jnp.int32, (1, B), 1)).astype(jnp.float32)
    hc0b = jnp.dot(oh, hc0, preferred_element_type=jnp.float32)
    hc1b = jnp.dot(oh, hc1, preferred_element_type=jnp.float32)
    x = aux[:, 3:4]
    u = u2[...]
    tce = x * hc0b + (1.0 - x) * hc1b
    att = (jnp.dot(u, watu[...], preferred_element_type=jnp.float32)
           + jnp.dot(tce, watt[...], preferred_element_type=jnp.float32)
           + cD[0:1, :])
    w0 = jnp.sum(att * hc0b, axis=1, keepdims=True)
    w1 = jnp.sum(att * hc1b, axis=1, keepdims=True)
    m = jnp.maximum(w0, w1)
    e0 = jnp.exp(w0 - m)
    e1 = jnp.exp(w1 - m)
    z = e0 + e1
    hs = (e0 / z) * hc0b + (e1 / z) * hc1b
    q[...] = (jnp.sum(hs * cD[1:2, :], axis=1, keepdims=True)
              + jnp.sum(u * cD[2:3, :], axis=1, keepdims=True)
              + jnp.sum(tce * cD[3:4, :], axis=1, keepdims=True)
              + cD[4:5, 0:1])


def _kd(u2, aux, batchp, s0, s1, den, watu, watt, cD):
    return pl.pallas_call(
        _kd_body,
        grid=(GRID,),
        in_specs=[_row_spec(HID), _row_spec(8), _row_spec(1),
                  _full_spec((B, HID)), _full_spec((B, HID)), _full_spec((B, 8)),
                  _full_spec((HID, HID)), _full_spec((HID, HID)),
                  _full_spec((5, HID))],
        out_specs=[_row_spec(1)],
        out_shape=[jax.ShapeDtypeStruct((N_PAD, 1), jnp.float32)],
    )(u2, aux, batchp, s0, s1, den, watu, watt, cD)[0]


@jax.jit
def kernel(x, edge_attr, params, edge_index, batch):
    layers = params["layers"]
    f32 = jnp.float32

    row = edge_index[0].astype(jnp.int32)
    col = edge_index[1].astype(jnp.int32)
    rowp = jnp.pad(row, (0, E_PAD - E), constant_values=N).reshape(EROWS, W)
    colp = jnp.pad(col, (0, E_PAD - E), constant_values=0).reshape(EROWS, W)
    eap = jnp.pad(edge_attr.astype(f32), (0, E_PAD - E))
    x2 = jnp.pad(x.astype(f32), ((0, N_PAD - N), (0, 0)))
    batchp = jnp.pad(batch.astype(jnp.int32), (0, N_PAD - N),
                     constant_values=B).reshape(N_PAD, 1)

    wb = jnp.zeros((128,), f32)
    wb = wb.at[0:3].set(jnp.stack([layers[h]["l3"]["W"][0, 0] for h in range(3)]))
    wb = wb.at[3:6].set(jnp.stack([layers[h]["l3"]["b"][0] for h in range(3)]))
    zeros8 = jnp.zeros((NPT, 8), f32)
    zeros32 = jnp.zeros((NPT, 32), f32)

    def cpack(lp):
        return jnp.stack([lp["l0"]["W"][:, 0], lp["l2"]["W"][:, 0],
                          lp["l0"]["b"] + lp["l1"]["b"] + lp["l2"]["b"]])

    p8 = _k1(rowp, eap, wb, zeros8)
    aux_in = _ka(p8[0], p8[1], x2, cpack(layers[0]))
    u0lo, u0hi, aux = aux_in

    ubig0 = jnp.concatenate([u0lo, u0hi], axis=0)
    s1o = _k2(colp, rowp, eap, ubig0, zeros32)
    u1lo, u1hi = _kb(s1o[0], s1o[1], aux, layers[1]["l1"]["W"].T, cpack(layers[1]))

    ubig1 = jnp.concatenate([u1lo, u1hi], axis=0)
    s2o = _k2(colp, rowp, eap, ubig1, zeros32)
    u2, s0, s1, den = _kc(s2o[0], s2o[1], aux, batchp,
                          layers[2]["l1"]["W"].T, cpack(layers[2]))

    Wa = params["attention"]["W"]
    Wl = params["last"]["W"][0]
    vs = params["space"]["W"].T @ Wl[:HID]
    va = params["action"]["W"].T @ Wl[HID:]
    cq = (params["space"]["b"] @ Wl[:HID] + params["action"]["b"] @ Wl[HID:]
          + params["last"]["b"][0])
    cD = jnp.stack([params["attention"]["b"], vs, va[:HID], va[HID:],
                    jnp.full((HID,), cq, f32)])
    q = _kd(u2, aux, batchp, s0, s1, den, Wa[:, :HID].T, Wa[:, HID:].T, cD)
    return q[:N]
